# baseline (speedup 1.0000x reference)
import jax
import jax.numpy as jnp
from jax import lax
from jax.experimental import pallas as pl
from jax.experimental.pallas import tpu as pltpu
from jax.experimental.pallas import tpu_sc as plsc

B = 16384
D_DRV, D_WEEK, D_TIME = 16, 3, 8
D_OUT = D_DRV + D_WEEK + D_TIME + 1
V_DRV, V_TIME = 24000, 1440
TJ_DRV = 188
TJ_TM = 12

DIST_MEAN = 10.0
DIST_STD = 5.0


def _body(drv_idx_hbm, wk_idx_hbm, tm_idx_hbm, dist_hbm,
          drv_tab_hbm, tm_tab_hbm, out_hbm,
          tab_v, idx_v, val_v, out_v, sem, sem_w):
    wid = lax.axis_index("s") * 2 + lax.axis_index("c")

    def gather_loop(off):
        writes = []
        for h in range(2):
            @plsc.parallel_loop(h * B // 2, (h + 1) * B // 2,
                                step=16, unroll=8)
            def _(i):
                idx = idx_v[pl.ds(i, 16)] + off
                out_v[i // 128, pl.ds(i % 128, 16)] = plsc.load_gather(
                    tab_v, [lax.shift_right_logical(idx, 7),
                            lax.bitwise_and(idx, 127)])

            writes.append(pltpu.async_copy(
                out_v.at[pl.ds(h * 64, 64), :],
                out_hbm.at[wid // 8, pl.ds(h * 64, 64), wid % 8, :],
                sem_w))
        for w in writes:
            w.wait()

    @pl.when(wid < D_DRV)
    def _():
        c1 = pltpu.async_copy(
            drv_tab_hbm.at[wid // 8, :, wid % 8, :], tab_v, sem)
        c2 = pltpu.async_copy(drv_idx_hbm, idx_v, sem)
        c1.wait()
        c2.wait()
        gather_loop(0)

    @pl.when(jnp.logical_and(wid >= D_DRV, wid < D_DRV + D_WEEK))
    def _():
        c1 = pltpu.async_copy(
            tm_tab_hbm.at[0, :, wid - D_DRV, :],
            tab_v.at[pl.ds(0, TJ_TM), :], sem)
        c2 = pltpu.async_copy(wk_idx_hbm, idx_v, sem)
        c1.wait()
        c2.wait()
        gather_loop(V_TIME)

    @pl.when(jnp.logical_and(wid >= D_DRV + D_WEEK, wid < D_OUT - 1))
    def _():
        c1 = pltpu.async_copy(
            tm_tab_hbm.at[0, :, wid - (D_DRV + D_WEEK), :],
            tab_v.at[pl.ds(0, TJ_TM), :], sem)
        c2 = pltpu.async_copy(tm_idx_hbm, idx_v, sem)
        c1.wait()
        c2.wait()
        gather_loop(0)

    @pl.when(wid == D_OUT - 1)
    def _():
        pltpu.async_copy(dist_hbm, val_v, sem).wait()

        @plsc.parallel_loop(0, B, step=16, unroll=8)
        def _(i):
            dv = val_v[pl.ds(i, 16)]
            out_v[i // 128, pl.ds(i % 128, 16)] = (
                dv * (1.0 / DIST_STD) - (DIST_MEAN / DIST_STD))

        pltpu.sync_copy(out_v, out_hbm.at[wid // 8, :, wid % 8, :])


def _tileblock(t, tj):
    f, v = t.shape
    fp = -(-f // 8) * 8
    t = jnp.pad(t, ((0, fp - f), (0, tj * 128 - v)))
    return t.reshape(fp // 8, 8, tj, 128).transpose(0, 2, 1, 3)


@jax.jit
def kernel(driverID, weekID, timeID, dist, driver_em, week_em, time_em):
    tm_plus = jnp.pad(time_em.T, ((0, 0), (0, 96)))
    tm_plus = lax.dynamic_update_slice(tm_plus, week_em.T, (0, V_TIME))
    mesh = plsc.VectorSubcoreMesh(core_axis_name="c", subcore_axis_name="s")
    k = pl.kernel(
        _body,
        out_type=jax.ShapeDtypeStruct((4, 128, 8, 128), jnp.float32),
        mesh=mesh,
        compiler_params=pltpu.CompilerParams(
            needs_layout_passes=False, use_tc_tiling_on_sc=False),
        scratch_types=[
            pltpu.VMEM((TJ_DRV, 128), jnp.float32),
            pltpu.VMEM((B,), jnp.int32),
            pltpu.VMEM((B,), jnp.float32),
            pltpu.VMEM((128, 128), jnp.float32),
            pltpu.SemaphoreType.DMA,
            pltpu.SemaphoreType.DMA,
        ],
    )
    out4 = k(driverID, weekID, timeID, dist,
             _tileblock(driver_em.T, TJ_DRV),
             tm_plus.reshape(1, 8, TJ_TM, 128).transpose(0, 2, 1, 3))
    out_t = out4.transpose(0, 2, 1, 3).reshape(32, B)
    return out_t[:D_OUT].T
